# baseline (speedup 1.0000x reference)
import jax
import jax.numpy as jnp
from jax.experimental import pallas as pl
from jax.experimental.pallas import tpu as pltpu

N = 100000
K = 128
BN = 2000
NB = N // BN


def _tc_body(x_ref, o_ref):
    x = x_ref[...]
    o_ref[...] = jnp.sum(x[:, 0, :] * x[:, 1, :] * x[:, 2, :], axis=-1)[None, None]


@jax.jit
def kernel(triples):
    out = pl.pallas_call(
        _tc_body,
        grid=(NB,),
        in_specs=[pl.BlockSpec((BN, 3, K), lambda i: (i, 0, 0))],
        out_specs=pl.BlockSpec((1, 1, BN), lambda i: (i, 0, 0)),
        out_shape=jax.ShapeDtypeStruct((NB, 1, BN), jnp.float32),
    )(triples)
    return out.reshape(N)
